# baseline (speedup 1.0000x reference)
import functools

import jax
import jax.numpy as jnp
from jax import lax
from jax.experimental import pallas as pl
from jax.experimental.pallas import tpu as pltpu
from jax.experimental.pallas import tpu_sc as plsc

NC = 2
NS = 16
NW = NC * NS
L = 16


@functools.lru_cache(maxsize=None)
def _make_sc_gather(B, D, n_head, n_tail):
    assert B % (NW * L) == 0
    b_per_w = B // NW

    mesh = plsc.VectorSubcoreMesh(core_axis_name="c", subcore_axis_name="s")

    scratch = [
        pltpu.VMEM((b_per_w,), jnp.int32),
        pltpu.VMEM((b_per_w,), jnp.int32),
        pltpu.VMEM((b_per_w,), jnp.int32),
        pltpu.VMEM((b_per_w,), jnp.float32),
        pltpu.VMEM((D, b_per_w), jnp.float32),
        pltpu.VMEM((D, b_per_w), jnp.float32),
        pltpu.SemaphoreType.DMA,
        pltpu.SemaphoreType.DMA,
    ]

    @functools.partial(
        pl.kernel,
        mesh=mesh,
        out_type=jax.ShapeDtypeStruct((D, B), jnp.float32),
        scratch_types=scratch,
        compiler_params=pltpu.CompilerParams(use_tc_tiling_on_sc=False),
    )
    def k(head_t_hbm, tail_t_hbm, ids_hbm, out_t_hbm,
          ids_v, idx_a, idx_b, mf, buf_a, buf_b, sem_a, sem_b):
        cid = lax.axis_index("c")
        sid = lax.axis_index("s")
        wid = cid * NS + sid
        base = wid * b_per_w
        pltpu.sync_copy(ids_hbm.at[pl.ds(base, b_per_w)], ids_v)

        iota = lax.iota(jnp.int32, L)
        for q in range(b_per_w // L):
            v = ids_v[pl.ds(q * L, L)]
            m = v >= n_head
            spread = (sid * (b_per_w // L) + q) * L + iota
            idx_a[pl.ds(q * L, L)] = jnp.where(m, spread, v)
            idx_b[pl.ds(q * L, L)] = jnp.where(m, v - n_head, spread % n_tail)
            mf[pl.ds(q * L, L)] = jnp.where(m, 1.0, 0.0)

        copies = []
        for c in range(D):
            copies.append(
                pltpu.async_copy(head_t_hbm.at[c].at[idx_a], buf_a.at[c], sem_a))
            copies.append(
                pltpu.async_copy(tail_t_hbm.at[c].at[idx_b], buf_b.at[c], sem_b))
        for cp in copies:
            cp.wait()

        def merge_row(c, carry):
            for q in range(b_per_w // L):
                a = buf_a[c, pl.ds(q * L, L)]
                b = buf_b[c, pl.ds(q * L, L)]
                f = mf[pl.ds(q * L, L)]
                buf_a[c, pl.ds(q * L, L)] = a + f * (b - a)
            return carry

        lax.fori_loop(0, D, merge_row, 0)

        pltpu.sync_copy(buf_a, out_t_hbm.at[:, pl.ds(base, b_per_w)])

    return k


def kernel(weight_head, trainable_buffer, input_ids):
    n_head, D = weight_head.shape
    n_tail = trainable_buffer.shape[0]
    B = input_ids.shape[0]
    k = _make_sc_gather(B, D, n_head, n_tail)
    out_t = k(jnp.swapaxes(weight_head, 0, 1),
              jnp.swapaxes(trainable_buffer, 0, 1),
              input_ids.astype(jnp.int32))
    return jnp.swapaxes(out_t, 0, 1)


# speedup vs baseline: 7.7505x; 7.7505x over previous
import functools

import jax
import jax.numpy as jnp
from jax import lax
from jax.experimental import pallas as pl
from jax.experimental.pallas import tpu as pltpu
from jax.experimental.pallas import tpu_sc as plsc

NC = 2
NS = 16
NW = NC * NS
L = 16
W = 128


@functools.lru_cache(maxsize=None)
def _make_sc_gather(B, D, n_head, n_tail):
    assert B % (NW * L) == 0 and D <= W
    b_per_w = B // NW
    n_rounds = 2
    b_per_r = b_per_w // n_rounds

    mesh = plsc.VectorSubcoreMesh(core_axis_name="c", subcore_axis_name="s")

    scratch = [
        pltpu.VMEM((b_per_w,), jnp.int32),
        pltpu.VMEM((n_rounds, b_per_r), jnp.int32),
        pltpu.VMEM((n_rounds, b_per_r), jnp.int32),
        pltpu.VMEM((b_per_w,), jnp.int32),
        pltpu.VMEM((b_per_r, W), jnp.float32),
        pltpu.VMEM((b_per_r, W), jnp.float32),
        pltpu.VMEM((b_per_w, D), jnp.float32),
        pltpu.SemaphoreType.DMA,
        pltpu.SemaphoreType.DMA,
    ]

    @functools.partial(
        pl.kernel,
        mesh=mesh,
        out_type=jax.ShapeDtypeStruct((B, D), jnp.float32),
        scratch_types=scratch,
        compiler_params=pltpu.CompilerParams(use_tc_tiling_on_sc=False,
                                             needs_layout_passes=False),
    )
    def k(head_hbm, tail_hbm, ids_hbm, out_hbm, ids_v, idx_a, idx_b,
          msk, buf_a, buf_b, blk, sem_a, sem_b):
        cid = lax.axis_index("c")
        sid = lax.axis_index("s")
        wid = cid * NS + sid
        base = wid * b_per_w
        pltpu.sync_copy(ids_hbm.at[pl.ds(base, b_per_w)], ids_v)

        iota = lax.iota(jnp.int32, L)
        for i in range(b_per_w // L):
            v = ids_v[pl.ds(i * L, L)]
            m = v >= n_head
            spread = (sid * (b_per_w // L) + i) * L + iota
            qr, qc = divmod(i * L, b_per_r)
            idx_a[qr, pl.ds(qc, L)] = jnp.where(m, spread, v)
            idx_b[qr, pl.ds(qc, L)] = jnp.where(m, v - n_head, spread % n_tail)
            msk[pl.ds(i * L, L)] = jnp.where(m, 1, 0)

        for r in range(n_rounds):
            ga = pltpu.async_copy(head_hbm.at[idx_a.at[r]], buf_a, sem_a)
            gb = pltpu.async_copy(tail_hbm.at[idx_b.at[r]], buf_b, sem_b)
            ga.wait()
            gb.wait()

            def assemble_col(c, carry, r=r):
                cv = jnp.zeros((L,), jnp.int32) + c
                for g in range(b_per_r // L):
                    s = r * b_per_r + g * L
                    rows = iota + g * L
                    a = plsc.load_gather(buf_a, [rows, cv])
                    b = plsc.load_gather(buf_b, [rows, cv])
                    mv = msk[pl.ds(s, L)]
                    vals = jnp.where(mv != 0, b, a)
                    plsc.store_scatter(blk, [s + iota, cv], vals)
                return carry

            lax.fori_loop(0, D, assemble_col, 0)

        pltpu.sync_copy(blk, out_hbm.at[pl.ds(base, b_per_w)])

    return k


def kernel(weight_head, trainable_buffer, input_ids):
    n_head, D = weight_head.shape
    n_tail = trainable_buffer.shape[0]
    B = input_ids.shape[0]
    k = _make_sc_gather(B, D, n_head, n_tail)
    return k(jnp.pad(weight_head, ((0, 0), (0, W - D))),
             jnp.pad(trainable_buffer, ((0, 0), (0, W - D))),
             input_ids.astype(jnp.int32))
